# Initial kernel scaffold; baseline (speedup 1.0000x reference)
import jax
import jax.numpy as jnp
from jax.experimental import pallas as pl


def kernel(pos_h, pos_t, pos_r, neg_h, neg_t, neg_r, ent_embeddings, rel_embeddings, ent_transfer, rel_transfer):
    raise NotImplementedError("write your pallas kernel here")


# speedup vs baseline: 28.8608x; 28.8608x over previous
import functools

import jax
import jax.numpy as jnp
from jax import lax
from jax.experimental import pallas as pl
from jax.experimental.pallas import tpu as pltpu
from jax.experimental.pallas import tpu_sc as plsc

NC = 2
NS = 16
L = 16
NW = NC * NS
B = 16384
NEG = 25
PAIRS = NEG + 1
H = 64
HV = H // L
ROWS_W = B // NW
CH_ROWS = 4
CP = CH_ROWS * PAIRS
NCH = ROWS_W // CH_ROWS
MARGIN = 1.0


def _sc_loss_call():
    mesh = plsc.VectorSubcoreMesh(
        core_axis_name="c", subcore_axis_name="s", num_cores=NC)

    @functools.partial(
        pl.kernel,
        mesh=mesh,
        compiler_params=pltpu.CompilerParams(use_tc_tiling_on_sc=False),
        out_type=jax.ShapeDtypeStruct((NW, L), jnp.float32),
        scratch_types=[
            pltpu.VMEM((NCH, CP), jnp.int32),
            pltpu.VMEM((NCH, CP), jnp.int32),
            pltpu.VMEM((NCH, CP), jnp.int32),
            pltpu.VMEM((2, CP, H), jnp.float32),
            pltpu.VMEM((2, CP, H), jnp.float32),
            pltpu.VMEM((2, CP, H), jnp.float32),
            pltpu.VMEM((2, CP, H), jnp.float32),
            pltpu.VMEM((2, CP, H), jnp.float32),
            pltpu.VMEM((2, CP, H), jnp.float32),
            pltpu.VMEM((L,), jnp.float32),
            pltpu.SemaphoreType.DMA,
            pltpu.SemaphoreType.DMA,
        ],
    )
    def sc_loss(h_hbm, t_hbm, r_hbm, ent_e, rel_e, ent_t, rel_t, out_hbm,
                hidx, tidx, ridx, he, ht, te, tt, rt, re, outv, sem0, sem1):
        wid = lax.axis_index("s") * NC + lax.axis_index("c")

        pltpu.sync_copy(h_hbm.at[wid], hidx)
        pltpu.sync_copy(t_hbm.at[wid], tidx)
        pltpu.sync_copy(r_hbm.at[wid], ridx)

        sems = (sem0, sem1)
        bufs = (he, ht, te, tt, rt, re)
        tabs = (ent_e, ent_t, ent_e, ent_t, rel_t, rel_e)
        idxs = (hidx, hidx, tidx, tidx, ridx, ridx)

        def start(g, slot):
            for buf, tab, ix in zip(bufs, tabs, idxs):
                pltpu.async_copy(tab.at[ix.at[g]], buf.at[slot], sems[slot])

        def drain(slot):
            for buf in bufs:
                pltpu.make_async_copy(
                    ent_e.at[pl.ds(0, CP)], buf.at[slot], sems[slot]).wait()

        def bsum(v):
            for step in (8, 4, 2, 1):
                idx = lax.iota(jnp.int32, L) ^ step
                v = v + v.at[idx].get(mode="promise_in_bounds")
            return v

        def pair_abs(slot, p):
            hev = [he[slot, p, pl.ds(i * L, L)] for i in range(HV)]
            htv = [ht[slot, p, pl.ds(i * L, L)] for i in range(HV)]
            tev = [te[slot, p, pl.ds(i * L, L)] for i in range(HV)]
            ttv = [tt[slot, p, pl.ds(i * L, L)] for i in range(HV)]
            rtv = [rt[slot, p, pl.ds(i * L, L)] for i in range(HV)]
            rev = [re[slot, p, pl.ds(i * L, L)] for i in range(HV)]
            q = hev[0] * htv[0] - tev[0] * ttv[0]
            for i in range(1, HV):
                q = q + hev[i] * htv[i] - tev[i] * ttv[i]
            d = bsum(q)
            return [jnp.abs(hev[i] - tev[i] + d * rtv[i] + rev[i])
                    for i in range(HV)]

        def compute(g, slot, partial):
            del g

            def row_body(rr, acc):
                base = rr * PAIRS
                pos = pair_abs(slot, base)

                def neg_body(k, neg):
                    for u in range(5):
                        nj = pair_abs(slot, base + 1 + k * 5 + u)
                        neg = tuple(neg[i] + nj[i] for i in range(HV))
                    return neg

                zero = jnp.zeros((L,), jnp.float32)
                neg = lax.fori_loop(0, NEG // 5, neg_body, (zero,) * HV)
                s_pos = bsum(pos[0] + pos[1] + pos[2] + pos[3])
                s_neg = bsum(neg[0] + neg[1] + neg[2] + neg[3])
                z = s_pos - s_neg * (1.0 / NEG) + MARGIN
                return acc + jnp.maximum(z, 0.0)

            return lax.fori_loop(0, CH_ROWS, row_body, partial)

        start(0, 0)

        def outer(i, partial):
            g = 2 * i
            start(g + 1, 1)
            drain(0)
            partial = compute(g, 0, partial)

            @pl.when(g + 2 < NCH)
            def _():
                start(g + 2, 0)

            drain(1)
            return compute(g + 1, 1, partial)

        partial = lax.fori_loop(0, NCH // 2, outer, jnp.zeros((L,), jnp.float32))
        outv[...] = partial
        pltpu.sync_copy(outv, out_hbm.at[wid])

    return sc_loss


_SC_LOSS = _sc_loss_call()


def kernel(pos_h, pos_t, pos_r, neg_h, neg_t, neg_r,
           ent_embeddings, rel_embeddings, ent_transfer, rel_transfer):
    h_idx = jnp.concatenate(
        [pos_h.astype(jnp.int32), neg_h.astype(jnp.int32)], axis=1
    ).reshape(NW, NCH, CP)
    t_idx = jnp.concatenate(
        [pos_t.astype(jnp.int32), neg_t.astype(jnp.int32)], axis=1
    ).reshape(NW, NCH, CP)
    r_idx = jnp.concatenate(
        [pos_r.astype(jnp.int32), neg_r.astype(jnp.int32)], axis=1
    ).reshape(NW, NCH, CP)
    partials = _SC_LOSS(h_idx, t_idx, r_idx, ent_embeddings, rel_embeddings,
                        ent_transfer, rel_transfer)
    return jnp.sum(partials[:, 0])
